# baseline (speedup 1.0000x reference)
import functools

import jax
import jax.numpy as jnp
from jax import lax
from jax.experimental import pallas as pl
from jax.experimental.pallas import tpu as pltpu
from jax.experimental.pallas import tpu_sc as plsc

_VERTICE_DIM = 15069
_N_WORKERS = 32
_L = 16


def _lip_distance_body(verts_hbm, faces_hbm, f2_hbm, w_hbm, out_hbm,
                       f2_v, w_v, fidx_v, v6_v, idx_v, g_v, out_v, sem):
    rows_per_w = out_hbm.shape[0] // _N_WORKERS
    n_chunks = rows_per_w // _L
    wid = lax.axis_index("s") * 2 + lax.axis_index("c")
    base = wid * rows_per_w

    pltpu.sync_copy(f2_hbm, f2_v)
    pltpu.sync_copy(w_hbm, w_v)

    lane = lax.iota(jnp.int32, 16)
    fvals = plsc.load_gather(f2_v, [lane // 3])
    fidx_v[...] = fvals * 3 + lane % 3
    pltpu.async_copy(faces_hbm.at[fidx_v], v6_v.at[pl.ds(8, 16)], sem).wait()

    for j in range(6):
        vj = plsc.load_gather(v6_v, [jnp.full((16,), 8 + j, jnp.int32)])
        colbase = vj * 3
        for d in range(3):
            t = 3 * j + d
            for c in range(n_chunks):
                rows = base + c * _L + lane
                idx_v[t, pl.ds(c * _L, _L)] = rows * _VERTICE_DIM + colbase + d

    copies = [pltpu.async_copy(verts_hbm.at[idx_v.at[t]], g_v.at[t], sem)
              for t in range(18)]
    for cp in copies:
        cp.wait()

    wj = [plsc.load_gather(w_v, [jnp.full((16,), 8 + j, jnp.int32)])
          for j in range(6)]
    for c in range(n_chunks):
        sl = pl.ds(c * _L, _L)
        dx = dy = dz = jnp.zeros((16,), jnp.float32)
        for j in range(6):
            dx = dx + wj[j] * g_v[3 * j + 0, sl]
            dy = dy + wj[j] * g_v[3 * j + 1, sl]
            dz = dz + wj[j] * g_v[3 * j + 2, sl]
        ss = dx * dx + dy * dy + dz * dz
        ssc = jnp.maximum(ss, jnp.float32(1e-30))
        bits = lax.bitcast_convert_type(ssc, jnp.int32)
        r = lax.bitcast_convert_type(0x5F3759DF - (bits >> 1), jnp.float32)
        for _ in range(3):
            r = r * (1.5 - 0.5 * ssc * r * r)
        out_v[sl] = ss * r * 1000.0
    pltpu.sync_copy(out_v, out_hbm.at[pl.ds(base, rows_per_w)])


def kernel(vertices, faces_tensor, lmk_faces_idx, lmk_bary_coords):
    B = vertices.shape[0]
    rows_per_w = B // _N_WORKERS

    verts_flat = vertices.reshape(-1)
    faces_flat = faces_tensor.astype(jnp.int32).reshape(-1)
    f2 = jnp.stack([lmk_faces_idx[45], lmk_faces_idx[49]]).astype(jnp.int32)
    f2 = jnp.concatenate([f2, jnp.zeros((14,), jnp.int32)])
    w6 = jnp.concatenate([lmk_bary_coords[45], -lmk_bary_coords[49]])
    w16 = jnp.concatenate([jnp.zeros((8,), jnp.float32), w6,
                           jnp.zeros((2,), jnp.float32)]).astype(jnp.float32)

    mesh = plsc.VectorSubcoreMesh(core_axis_name="c", subcore_axis_name="s")
    run = functools.partial(
        pl.kernel,
        out_type=jax.ShapeDtypeStruct((B,), jnp.float32),
        mesh=mesh,
        compiler_params=pltpu.CompilerParams(needs_layout_passes=False),
        scratch_types=[
            pltpu.VMEM((16,), jnp.int32),
            pltpu.VMEM((16,), jnp.float32),
            pltpu.VMEM((16,), jnp.int32),
            pltpu.VMEM((32,), jnp.int32),
            pltpu.VMEM((18, rows_per_w), jnp.int32),
            pltpu.VMEM((18, rows_per_w), jnp.float32),
            pltpu.VMEM((rows_per_w,), jnp.float32),
            pltpu.SemaphoreType.DMA,
        ],
    )(_lip_distance_body)
    return run(verts_flat, faces_flat, f2, w16)


# speedup vs baseline: 10.5951x; 10.5951x over previous
import functools

import jax
import jax.numpy as jnp
from jax import lax
from jax.experimental import pallas as pl
from jax.experimental.pallas import tpu as pltpu
from jax.experimental.pallas import tpu_sc as plsc

_D = 15069
_LAST_TILE = ((_D - 1) // 128) * 128
_N_WORKERS = 32
_L = 16


def _body(verts_hbm, faces_hbm, f2_hbm, w_hbm, out_hbm,
          f2_v, w_v, fidx_v, v6_v, vt_v, out_v, sem):
    B = out_hbm.shape[0]
    rows_per_w = B // _N_WORKERS
    n_chunks = rows_per_w // _L
    wid = lax.axis_index("s") * 2 + lax.axis_index("c")
    base = wid * rows_per_w

    pltpu.sync_copy(f2_hbm, f2_v)
    pltpu.sync_copy(w_hbm, w_v)

    lane = lax.iota(jnp.int32, 16)
    fvals = plsc.load_gather(f2_v, [lane // 3])
    fidx_v[...] = fvals * 3 + lane % 3
    pltpu.async_copy(faces_hbm.at[fidx_v], v6_v.at[pl.ds(8, 16)], sem).wait()

    v6vec = v6_v[pl.ds(8, 16)]
    wvec = w_v[...]
    tbs, tb2s, offs, wjs = [], [], [], []
    for j in range(6):
        col = v6vec[j] * 3
        tb = pl.multiple_of((col // 128) * 128, 128)
        tb2 = pl.multiple_of(jnp.minimum(tb + 128, _LAST_TILE), 128)
        tbs.append(tb)
        tb2s.append(tb2)
        offs.append(col - tb)
        wjs.append(plsc.load_gather(w_v, [jnp.full((16,), 8 + j, jnp.int32)]))

    for c in range(n_chunks):
        rs = base + c * _L
        copies = []
        for j in range(6):
            copies.append(pltpu.async_copy(
                verts_hbm.at[pl.ds(rs, _L), pl.ds(tbs[j], 128)],
                vt_v.at[2 * j], sem))
            copies.append(pltpu.async_copy(
                verts_hbm.at[pl.ds(rs, _L), pl.ds(tb2s[j], 128)],
                vt_v.at[2 * j + 1], sem))
        for cp in copies:
            cp.wait()
        dx = dy = dz = jnp.zeros((16,), jnp.float32)
        for j in range(6):
            for d in range(3):
                od = offs[j] + d
                tsel = jnp.zeros((16,), jnp.int32) + (2 * j + od // 128)
                csel = jnp.zeros((16,), jnp.int32) + (od % 128)
                val = plsc.load_gather(vt_v, [tsel, lane, csel])
                if d == 0:
                    dx = dx + wjs[j] * val
                elif d == 1:
                    dy = dy + wjs[j] * val
                else:
                    dz = dz + wjs[j] * val
        ss = dx * dx + dy * dy + dz * dz
        ssc = jnp.maximum(ss, jnp.float32(1e-30))
        bits = lax.bitcast_convert_type(ssc, jnp.int32)
        r = lax.bitcast_convert_type(0x5F3759DF - (bits >> 1), jnp.float32)
        for _ in range(3):
            r = r * (1.5 - 0.5 * ssc * r * r)
        out_v[pl.ds(c * _L, _L)] = ss * r * 1000.0
    pltpu.sync_copy(out_v, out_hbm.at[pl.ds(base, rows_per_w)])


def kernel(vertices, faces_tensor, lmk_faces_idx, lmk_bary_coords):
    B = vertices.shape[0]
    rows_per_w = B // _N_WORKERS

    faces_flat = faces_tensor.astype(jnp.int32).reshape(-1)
    f2 = jnp.stack([lmk_faces_idx[45], lmk_faces_idx[49]]).astype(jnp.int32)
    f2 = jnp.concatenate([f2, jnp.zeros((14,), jnp.int32)])
    w6 = jnp.concatenate([lmk_bary_coords[45], -lmk_bary_coords[49]])
    w16 = jnp.concatenate([jnp.zeros((8,), jnp.float32), w6,
                           jnp.zeros((2,), jnp.float32)]).astype(jnp.float32)

    mesh = plsc.VectorSubcoreMesh(core_axis_name="c", subcore_axis_name="s")
    run = functools.partial(
        pl.kernel,
        out_type=jax.ShapeDtypeStruct((B,), jnp.float32),
        mesh=mesh,
        compiler_params=pltpu.CompilerParams(
            needs_layout_passes=False, disable_bounds_checks=True),
        scratch_types=[
            pltpu.VMEM((16,), jnp.int32),
            pltpu.VMEM((16,), jnp.float32),
            pltpu.VMEM((16,), jnp.int32),
            pltpu.VMEM((32,), jnp.int32),
            pltpu.VMEM((12, _L, 128), jnp.float32),
            pltpu.VMEM((rows_per_w,), jnp.float32),
            pltpu.SemaphoreType.DMA,
        ],
    )(_body)
    return run(vertices, faces_flat, f2, w16)
